# baseline (speedup 1.0000x reference)
import jax
import jax.numpy as jnp
from jax.experimental import pallas as pl
from jax.experimental.pallas import tpu as pltpu

DZ = 32
NSPLIT = 2

EPS1 = -0.02830461598932743


def _body(x_ref, act_ref,
          we1_ref, be1_ref, we2_ref, be2_ref, we3_ref, be3_ref,
          wt1_ref, bt1_ref, wv_ref, bv_ref, wr_ref, br_ref, wo_ref, bo_ref,
          bmat_ref, wd1_ref, bd1_ref, wd2_ref, bd2_ref, wd3_ref, bd3_ref,
          out_ref,
          swe1, swe2, swe3, swt1, swvro, sbmat, swd1, swd2, swd3):
    f32 = jnp.float32
    bf16 = jnp.bfloat16

    @pl.when(pl.program_id(0) == 0)
    def _load_weights():
        swe1[...] = we1_ref[...].astype(bf16)
        swe2[...] = we2_ref[...].astype(bf16)
        swe3[...] = we3_ref[...].astype(bf16)
        swt1[...] = wt1_ref[...].astype(bf16)
        swvro[:, :DZ] = wv_ref[...].astype(bf16)
        swvro[:, DZ:2 * DZ] = wr_ref[...].astype(bf16)
        swvro[:, 2 * DZ:] = wo_ref[...].astype(bf16)
        sbmat[...] = bmat_ref[...].astype(bf16)
        swd1[...] = wd1_ref[...].astype(bf16)
        swd2[...] = wd2_ref[...].astype(bf16)
        swd3[...] = wd3_ref[...].astype(bf16)

    def mm(a, w_ref, prec=jnp.float32):
        return jax.lax.dot_general(
            a.astype(bf16), w_ref[...],
            (((1,), (0,)), ((), ())),
            preferred_element_type=prec)

    bm = x_ref.shape[0]
    sub = bm // NSPLIT
    for k in range(NSPLIT):
        rows = pl.ds(k * sub, sub)
        b1 = be1_ref[...].astype(bf16)
        b2 = be2_ref[...].astype(bf16)
        h1 = jnp.maximum(mm(x_ref[rows, :], swe1).astype(bf16) + b1, 0)
        h2 = jnp.maximum(mm(h1, swe2).astype(bf16) + b2, 0)
        ml = mm(h2, swe3) + be3_ref[...]
        mean = ml[:, :DZ]
        logvar = ml[:, DZ:]
        z = mean + EPS1 * jnp.exp(0.5 * logvar)

        h = jnp.maximum(mm(z, swt1) + bt1_ref[...], 0.0)
        vro = mm(h, swvro)
        v = vro[:, :DZ] + bv_ref[...]
        r = vro[:, DZ:2 * DZ] + br_ref[...]
        o = vro[:, 2 * DZ:] + bo_ref[...]
        s = jnp.sum(r * z, axis=1, keepdims=True)
        znp = z + v * s + mm(act_ref[rows, :], sbmat) + o

        c1 = bd1_ref[...].astype(bf16)
        c2 = bd2_ref[...].astype(bf16)
        d1 = jnp.maximum(mm(znp, swd1).astype(bf16) + c1, 0)
        d2 = jnp.maximum(mm(d1, swd2).astype(bf16) + c2, 0)
        out_ref[rows, :] = jax.nn.sigmoid(mm(d2, swd3) + bd3_ref[...])


def kernel(x, action, x_next, We1, be1, We2, be2, We3, be3, Wd1, bd1, Wd2,
           bd2, Wd3, bd3, Wt1, bt1, Wv, bv, Wr, br, Wo, bo, Bmat, codebook):
    B, DIN = x.shape
    H = We1.shape[1]
    HT = Wt1.shape[1]
    DU = action.shape[1]
    BM = 2048

    bf16 = jnp.bfloat16
    row = lambda i: (i, 0)
    rep = lambda i: (0, 0)
    rep1 = lambda i: (0,)

    def wspec(shape):
        return pl.BlockSpec(shape, rep)

    def bspec(n):
        return pl.BlockSpec((n,), rep1)

    grid = B // BM
    out = pl.pallas_call(
        _body,
        grid=(grid,),
        in_specs=[
            pl.BlockSpec((BM, DIN), row),
            pl.BlockSpec((BM, DU), row),
            wspec((DIN, H)), bspec(H),
            wspec((H, H)), bspec(H),
            wspec((H, 2 * DZ)), bspec(2 * DZ),
            wspec((DZ, HT)), bspec(HT),
            wspec((HT, DZ)), bspec(DZ),
            wspec((HT, DZ)), bspec(DZ),
            wspec((HT, DZ)), bspec(DZ),
            wspec((DU, DZ)),
            wspec((DZ, H)), bspec(H),
            wspec((H, H)), bspec(H),
            wspec((H, DIN)), bspec(DIN),
        ],
        out_specs=pl.BlockSpec((BM, DIN), row),
        out_shape=jax.ShapeDtypeStruct((B, DIN), jnp.float32),
        scratch_shapes=[
            pltpu.VMEM((DIN, H), bf16),
            pltpu.VMEM((H, H), bf16),
            pltpu.VMEM((H, 2 * DZ), bf16),
            pltpu.VMEM((DZ, HT), bf16),
            pltpu.VMEM((HT, 3 * DZ), bf16),
            pltpu.VMEM((DU, DZ), bf16),
            pltpu.VMEM((DZ, H), bf16),
            pltpu.VMEM((H, H), bf16),
            pltpu.VMEM((H, DIN), bf16),
        ],
        compiler_params=pltpu.CompilerParams(
            dimension_semantics=("arbitrary",),
        ),
    )(
        x, action,
        We1, be1, We2, be2, We3, be3,
        Wt1, bt1, Wv, bv, Wr, br, Wo, bo, Bmat,
        Wd1, bd1, Wd2, bd2, Wd3, bd3,
    )
    return out


# speedup vs baseline: 1.0295x; 1.0295x over previous
import jax
import jax.numpy as jnp
from jax.experimental import pallas as pl
from jax.experimental.pallas import tpu as pltpu

DZ = 32
NSPLIT = 1

EPS1 = -0.02830461598932743


def _body(x_ref, act_ref,
          we1_ref, be1_ref, we2_ref, be2_ref, we3_ref, be3_ref,
          wt1_ref, bt1_ref, wv_ref, bv_ref, wr_ref, br_ref, wo_ref, bo_ref,
          bmat_ref, wd1_ref, bd1_ref, wd2_ref, bd2_ref, wd3_ref, bd3_ref,
          out_ref,
          swe1, swe2, swe3, swt1, swvro, sbmat, swd1, swd2, swd3):
    f32 = jnp.float32
    bf16 = jnp.bfloat16

    @pl.when(pl.program_id(0) == 0)
    def _load_weights():
        swe1[...] = we1_ref[...].astype(bf16)
        swe2[...] = we2_ref[...].astype(bf16)
        swe3[...] = we3_ref[...].astype(bf16)
        swt1[...] = wt1_ref[...].astype(bf16)
        swvro[:, :DZ] = wv_ref[...].astype(bf16)
        swvro[:, DZ:2 * DZ] = wr_ref[...].astype(bf16)
        swvro[:, 2 * DZ:] = wo_ref[...].astype(bf16)
        sbmat[...] = bmat_ref[...].astype(bf16)
        swd1[...] = wd1_ref[...].astype(bf16)
        swd2[...] = wd2_ref[...].astype(bf16)
        swd3[...] = wd3_ref[...].astype(bf16)

    def mm(a, w_ref, prec=jnp.float32):
        return jax.lax.dot_general(
            a.astype(bf16), w_ref[...],
            (((1,), (0,)), ((), ())),
            preferred_element_type=prec)

    bm = x_ref.shape[0]
    sub = bm // NSPLIT
    for k in range(NSPLIT):
        rows = pl.ds(k * sub, sub)
        b1 = be1_ref[...].astype(bf16)
        b2 = be2_ref[...].astype(bf16)
        h1 = jnp.maximum(mm(x_ref[rows, :], swe1).astype(bf16) + b1, 0)
        h2 = jnp.maximum(mm(h1, swe2).astype(bf16) + b2, 0)
        ml = mm(h2, swe3) + be3_ref[...]
        mean = ml[:, :DZ]
        logvar = ml[:, DZ:]
        z = mean + EPS1 * jnp.exp(0.5 * logvar)

        h = jnp.maximum(mm(z, swt1) + bt1_ref[...], 0.0)
        vro = mm(h, swvro)
        v = vro[:, :DZ] + bv_ref[...]
        r = vro[:, DZ:2 * DZ] + br_ref[...]
        o = vro[:, 2 * DZ:] + bo_ref[...]
        s = jnp.sum(r * z, axis=1, keepdims=True)
        znp = z + v * s + mm(act_ref[rows, :], sbmat) + o

        c1 = bd1_ref[...].astype(bf16)
        c2 = bd2_ref[...].astype(bf16)
        d1 = jnp.maximum(mm(znp, swd1).astype(bf16) + c1, 0)
        d2 = jnp.maximum(mm(d1, swd2).astype(bf16) + c2, 0)
        out_ref[rows, :] = jax.nn.sigmoid(mm(d2, swd3) + bd3_ref[...])


def kernel(x, action, x_next, We1, be1, We2, be2, We3, be3, Wd1, bd1, Wd2,
           bd2, Wd3, bd3, Wt1, bt1, Wv, bv, Wr, br, Wo, bo, Bmat, codebook):
    B, DIN = x.shape
    H = We1.shape[1]
    HT = Wt1.shape[1]
    DU = action.shape[1]
    BM = 1024

    bf16 = jnp.bfloat16
    row = lambda i: (i, 0)
    rep = lambda i: (0, 0)
    rep1 = lambda i: (0,)

    def wspec(shape):
        return pl.BlockSpec(shape, rep)

    def bspec(n):
        return pl.BlockSpec((n,), rep1)

    grid = B // BM
    out = pl.pallas_call(
        _body,
        grid=(grid,),
        in_specs=[
            pl.BlockSpec((BM, DIN), row),
            pl.BlockSpec((BM, DU), row),
            wspec((DIN, H)), bspec(H),
            wspec((H, H)), bspec(H),
            wspec((H, 2 * DZ)), bspec(2 * DZ),
            wspec((DZ, HT)), bspec(HT),
            wspec((HT, DZ)), bspec(DZ),
            wspec((HT, DZ)), bspec(DZ),
            wspec((HT, DZ)), bspec(DZ),
            wspec((DU, DZ)),
            wspec((DZ, H)), bspec(H),
            wspec((H, H)), bspec(H),
            wspec((H, DIN)), bspec(DIN),
        ],
        out_specs=pl.BlockSpec((BM, DIN), row),
        out_shape=jax.ShapeDtypeStruct((B, DIN), jnp.float32),
        scratch_shapes=[
            pltpu.VMEM((DIN, H), bf16),
            pltpu.VMEM((H, H), bf16),
            pltpu.VMEM((H, 2 * DZ), bf16),
            pltpu.VMEM((DZ, HT), bf16),
            pltpu.VMEM((HT, 3 * DZ), bf16),
            pltpu.VMEM((DU, DZ), bf16),
            pltpu.VMEM((DZ, H), bf16),
            pltpu.VMEM((H, H), bf16),
            pltpu.VMEM((H, DIN), bf16),
        ],
        compiler_params=pltpu.CompilerParams(
            dimension_semantics=("arbitrary",),
        ),
    )(
        x, action,
        We1, be1, We2, be2, We3, be3,
        Wt1, bt1, Wv, bv, Wr, br, Wo, bo, Bmat,
        Wd1, bd1, Wd2, bd2, Wd3, bd3,
    )
    return out


# speedup vs baseline: 1.0504x; 1.0203x over previous
import jax
import jax.numpy as jnp
from jax.experimental import pallas as pl
from jax.experimental.pallas import tpu as pltpu

DZ = 32
NSPLIT = 1

EPS1 = -0.02830461598932743


def _body(x_ref, act_ref,
          we1_ref, be1_ref, we2_ref, be2_ref, we3_ref, be3_ref,
          wt1_ref, bt1_ref, wv_ref, bv_ref, wr_ref, br_ref, wo_ref, bo_ref,
          bmat_ref, wd1_ref, bd1_ref, wd2_ref, bd2_ref, wd3_ref, bd3_ref,
          out_ref,
          swe1, swe2, swe3, swt1, swvro, sbmat, swd1, swd2, swd3):
    f32 = jnp.float32
    bf16 = jnp.bfloat16

    @pl.when(pl.program_id(0) == 0)
    def _load_weights():
        swe1[...] = we1_ref[...].astype(bf16)
        swe2[...] = we2_ref[...].astype(bf16)
        swe3[...] = we3_ref[...].astype(bf16)
        swt1[...] = wt1_ref[...].astype(bf16)
        swvro[:, :DZ] = wv_ref[...].astype(bf16)
        swvro[:, DZ:2 * DZ] = wr_ref[...].astype(bf16)
        swvro[:, 2 * DZ:] = wo_ref[...].astype(bf16)
        sbmat[...] = bmat_ref[...].astype(bf16)
        swd1[...] = wd1_ref[...].astype(bf16)
        swd2[...] = wd2_ref[...].astype(bf16)
        swd3[...] = wd3_ref[...].astype(bf16)

    def mm(a, w_ref, prec=jnp.float32):
        return jax.lax.dot_general(
            a.astype(bf16), w_ref[...],
            (((1,), (0,)), ((), ())),
            preferred_element_type=prec)

    bm = x_ref.shape[0]
    sub = bm // NSPLIT
    for k in range(NSPLIT):
        rows = pl.ds(k * sub, sub)
        b1 = be1_ref[...].astype(bf16)
        b2 = be2_ref[...].astype(bf16)
        h1 = jnp.maximum(mm(x_ref[rows, :], swe1).astype(bf16) + b1, 0)
        h2 = jnp.maximum(mm(h1, swe2).astype(bf16) + b2, 0)
        ml = mm(h2, swe3) + be3_ref[...]
        mean = ml[:, :DZ]
        logvar = ml[:, DZ:]
        z = mean + EPS1 * jnp.exp(0.5 * logvar)

        h = jnp.maximum(mm(z, swt1) + bt1_ref[...], 0.0)
        vro = mm(h, swvro)
        v = vro[:, :DZ] + bv_ref[...]
        r = vro[:, DZ:2 * DZ] + br_ref[...]
        o = vro[:, 2 * DZ:] + bo_ref[...]
        s = jnp.sum(r * z, axis=1, keepdims=True)
        znp = z + v * s + mm(act_ref[rows, :], sbmat) + o

        c1 = bd1_ref[...].astype(bf16)
        c2 = bd2_ref[...].astype(bf16)
        d1 = jnp.maximum(mm(znp, swd1).astype(bf16) + c1, 0)
        d2 = jnp.maximum(mm(d1, swd2).astype(bf16) + c2, 0)
        t = mm(d2, swd3) + bd3_ref[...]
        out_ref[rows, :] = 0.5 * jnp.tanh(0.5 * t) + 0.5


def kernel(x, action, x_next, We1, be1, We2, be2, We3, be3, Wd1, bd1, Wd2,
           bd2, Wd3, bd3, Wt1, bt1, Wv, bv, Wr, br, Wo, bo, Bmat, codebook):
    B, DIN = x.shape
    H = We1.shape[1]
    HT = Wt1.shape[1]
    DU = action.shape[1]
    BM = 1024

    bf16 = jnp.bfloat16
    row = lambda i: (i, 0)
    rep = lambda i: (0, 0)
    rep1 = lambda i: (0,)

    def wspec(shape):
        return pl.BlockSpec(shape, rep)

    def bspec(n):
        return pl.BlockSpec((n,), rep1)

    grid = B // BM
    out = pl.pallas_call(
        _body,
        grid=(grid,),
        in_specs=[
            pl.BlockSpec((BM, DIN), row),
            pl.BlockSpec((BM, DU), row),
            wspec((DIN, H)), bspec(H),
            wspec((H, H)), bspec(H),
            wspec((H, 2 * DZ)), bspec(2 * DZ),
            wspec((DZ, HT)), bspec(HT),
            wspec((HT, DZ)), bspec(DZ),
            wspec((HT, DZ)), bspec(DZ),
            wspec((HT, DZ)), bspec(DZ),
            wspec((DU, DZ)),
            wspec((DZ, H)), bspec(H),
            wspec((H, H)), bspec(H),
            wspec((H, DIN)), bspec(DIN),
        ],
        out_specs=pl.BlockSpec((BM, DIN), row),
        out_shape=jax.ShapeDtypeStruct((B, DIN), jnp.float32),
        scratch_shapes=[
            pltpu.VMEM((DIN, H), bf16),
            pltpu.VMEM((H, H), bf16),
            pltpu.VMEM((H, 2 * DZ), bf16),
            pltpu.VMEM((DZ, HT), bf16),
            pltpu.VMEM((HT, 3 * DZ), bf16),
            pltpu.VMEM((DU, DZ), bf16),
            pltpu.VMEM((DZ, H), bf16),
            pltpu.VMEM((H, H), bf16),
            pltpu.VMEM((H, DIN), bf16),
        ],
        compiler_params=pltpu.CompilerParams(
            dimension_semantics=("arbitrary",),
        ),
    )(
        x, action,
        We1, be1, We2, be2, We3, be3,
        Wt1, bt1, Wv, bv, Wr, br, Wo, bo, Bmat,
        Wd1, bd1, Wd2, bd2, Wd3, bd3,
    )
    return out
